# Initial kernel scaffold; baseline (speedup 1.0000x reference)
import functools

import jax
import jax.numpy as jnp
from jax import lax
from jax.experimental import pallas as pl
from jax.experimental.pallas import tpu as pltpu
from jax.experimental.pallas import tpu_sc as plsc

N = 10000
E = 320000
D = 128
NC = 2
NS = 16
NW = NC * NS
E_PER_W = E // NW
CHUNK = 400
N_CHUNKS = E_PER_W // CHUNK
ROW_BLK = 624
LANES = 16


def _mm_kernel(x_ref, w_ref, o_ref):
    o_ref[...] = jnp.dot(x_ref[...], w_ref[...],
                         preferred_element_type=jnp.float32)


def _matmul(x, w):
    return pl.pallas_call(
        _mm_kernel,
        out_shape=jax.ShapeDtypeStruct((N, D), jnp.float32),
    )(x, w)


def _comb_mm_kernel(p_ref, w_ref, o_ref):
    s = p_ref[0] + p_ref[1]
    s = jnp.where(s >= 0, s, 0.01 * s)
    o_ref[...] = jnp.dot(s, w_ref[...], preferred_element_type=jnp.float32)


def _comb_matmul(p, w):
    return pl.pallas_call(
        _comb_mm_kernel,
        out_shape=jax.ShapeDtypeStruct((N, D), jnp.float32),
    )(p, w)


def _comb_kernel(p_ref, o_ref):
    s = p_ref[0] + p_ref[1]
    o_ref[...] = jnp.where(s >= 0, s, 0.01 * s)


def _combine(p):
    return pl.pallas_call(
        _comb_kernel,
        out_shape=jax.ShapeDtypeStruct((N, D), jnp.float32),
    )(p)


_sc_mesh = plsc.VectorSubcoreMesh(
    core_axis_name="c", subcore_axis_name="s", num_cores=NC, num_subcores=NS)


@functools.partial(
    pl.kernel,
    out_type=jax.ShapeDtypeStruct((NC, N, D), jnp.float32),
    mesh=_sc_mesh,
    scratch_types=[
        pltpu.VMEM_SHARED((N, D), jnp.float32),
        pltpu.VMEM((CHUNK, D), jnp.float32),
        pltpu.VMEM((CHUNK,), jnp.int32),
        pltpu.VMEM((CHUNK,), jnp.int32),
        pltpu.VMEM((CHUNK,), jnp.float32),
        pltpu.SemaphoreType.DMA,
    ],
)
def _sc_scatter(h_hbm, src_hbm, dst_hbm, ew_hbm, out_hbm,
                acc, rows, sidx, didx, ew, sem):
    c = lax.axis_index("c")
    s = lax.axis_index("s")

    @pl.loop(0, CHUNK)
    def _zero_rows(i):
        for j in range(D // LANES):
            rows[i, pl.ds(j * LANES, LANES)] = jnp.zeros((LANES,), jnp.float32)

    base_row = s * ROW_BLK
    pltpu.sync_copy(rows.at[pl.ds(0, 400)], acc.at[pl.ds(base_row, 400)])
    pltpu.sync_copy(rows.at[pl.ds(0, 240)], acc.at[pl.ds(base_row + 400, 240)])
    plsc.subcore_barrier()

    ebase = (c * NS + s) * E_PER_W

    @pl.loop(0, N_CHUNKS)
    def _chunk(k):
        eb = ebase + k * CHUNK
        pltpu.sync_copy(src_hbm.at[pl.ds(eb, CHUNK)], sidx)
        pltpu.sync_copy(dst_hbm.at[pl.ds(eb, CHUNK)], didx)
        pltpu.sync_copy(ew_hbm.at[pl.ds(eb, CHUNK)], ew)
        pltpu.async_copy(h_hbm.at[sidx], rows, sem).wait()

        @pl.loop(0, CHUNK)
        def _scale(e):
            w = ew[e]
            for j in range(D // LANES):
                slc = pl.ds(j * LANES, LANES)
                rows[e, slc] = rows[e, slc] * w

        pltpu.async_copy(rows, acc.at[didx], sem, add=True).wait()

    plsc.subcore_barrier()
    pltpu.sync_copy(acc.at[pl.ds(base_row, 400)],
                    out_hbm.at[c].at[pl.ds(base_row, 400)])
    pltpu.sync_copy(acc.at[pl.ds(base_row + 400, 240)],
                    out_hbm.at[c].at[pl.ds(base_row + 400, 240)])


def kernel(x, edge_index, edge_w, W0, W1):
    src = edge_index[0].astype(jnp.int32)
    dst = edge_index[1].astype(jnp.int32)
    h0 = _matmul(x, W0)
    p0 = _sc_scatter(h0, src, dst, edge_w)
    h1 = _comb_matmul(p0, W1)
    p1 = _sc_scatter(h1, src, dst, edge_w)
    return _combine(p1)


# speedup vs baseline: 3.9164x; 3.9164x over previous
import functools

import jax
import jax.numpy as jnp
from jax import lax
from jax.experimental import pallas as pl
from jax.experimental.pallas import tpu as pltpu
from jax.experimental.pallas import tpu_sc as plsc

N = 10000
E = 320000
D = 128
NC = 2
NS = 16
NW = NC * NS
E_PER_W = E // NW
CHUNK = 80
N_CHUNKS = E_PER_W // CHUNK
ROW_BLK = 624
LANES = 16


def _mm_kernel(x_ref, w_ref, o_ref):
    o_ref[...] = jnp.dot(x_ref[...], w_ref[...],
                         preferred_element_type=jnp.float32)


def _matmul(x, w):
    return pl.pallas_call(
        _mm_kernel,
        out_shape=jax.ShapeDtypeStruct((N, D), jnp.float32),
    )(x, w)


def _comb_mm_kernel(p_ref, w_ref, o_ref):
    s = p_ref[0] + p_ref[1]
    s = jnp.where(s >= 0, s, 0.01 * s)
    o_ref[...] = jnp.dot(s, w_ref[...], preferred_element_type=jnp.float32)


def _comb_matmul(p, w):
    return pl.pallas_call(
        _comb_mm_kernel,
        out_shape=jax.ShapeDtypeStruct((N, D), jnp.float32),
    )(p, w)


def _comb_kernel(p_ref, o_ref):
    s = p_ref[0] + p_ref[1]
    o_ref[...] = jnp.where(s >= 0, s, 0.01 * s)


def _combine(p):
    return pl.pallas_call(
        _comb_kernel,
        out_shape=jax.ShapeDtypeStruct((N, D), jnp.float32),
    )(p)


_sc_mesh = plsc.VectorSubcoreMesh(
    core_axis_name="c", subcore_axis_name="s", num_cores=NC, num_subcores=NS)


@functools.partial(
    pl.kernel,
    out_type=jax.ShapeDtypeStruct((NC, N, D), jnp.float32),
    mesh=_sc_mesh,
    scratch_types=[
        pltpu.VMEM_SHARED((N, D), jnp.float32),
        pltpu.VMEM((CHUNK, D), jnp.float32),
        pltpu.VMEM((CHUNK,), jnp.int32),
        pltpu.VMEM((CHUNK,), jnp.int32),
        pltpu.VMEM((CHUNK,), jnp.float32),
        pltpu.SemaphoreType.DMA,
    ],
)
def _sc_scatter(h_hbm, src_hbm, dst_hbm, ew_hbm, out_hbm,
                acc, rows, sidx, didx, ew, sem):
    c = lax.axis_index("c")
    s = lax.axis_index("s")

    @pl.loop(0, CHUNK)
    def _zero_rows(i):
        for j in range(D // LANES):
            rows[i, pl.ds(j * LANES, LANES)] = jnp.zeros((LANES,), jnp.float32)

    base_row = s * ROW_BLK
    for off in range(0, 640, CHUNK):
        pltpu.sync_copy(rows, acc.at[pl.ds(base_row + off, CHUNK)])
    plsc.subcore_barrier()

    ebase = (c * NS + s) * E_PER_W

    @pl.loop(0, N_CHUNKS)
    def _chunk(k):
        eb = ebase + k * CHUNK
        pltpu.sync_copy(src_hbm.at[pl.ds(eb, CHUNK)], sidx)
        pltpu.sync_copy(dst_hbm.at[pl.ds(eb, CHUNK)], didx)
        pltpu.sync_copy(ew_hbm.at[pl.ds(eb, CHUNK)], ew)
        pltpu.async_copy(h_hbm.at[sidx], rows, sem).wait()

        @pl.loop(0, CHUNK, step=LANES)
        def _scale(b):
            wv = ew[pl.ds(b, LANES)]
            for l in range(LANES):
                w = wv[l]
                for j in range(D // LANES):
                    slc = pl.ds(j * LANES, LANES)
                    rows[b + l, slc] = rows[b + l, slc] * w

        pltpu.async_copy(rows, acc.at[didx], sem, add=True).wait()

    plsc.subcore_barrier()
    pltpu.sync_copy(acc.at[pl.ds(base_row, 640)],
                    out_hbm.at[c].at[pl.ds(base_row, 640)])


def kernel(x, edge_index, edge_w, W0, W1):
    src = edge_index[0].astype(jnp.int32)
    dst = edge_index[1].astype(jnp.int32)
    h0 = _matmul(x, W0)
    p0 = _sc_scatter(h0, src, dst, edge_w)
    h1 = _comb_matmul(p0, W1)
    p1 = _sc_scatter(h1, src, dst, edge_w)
    return _combine(p1)


# speedup vs baseline: 6.2833x; 1.6044x over previous
import functools

import jax
import jax.numpy as jnp
from jax import lax
from jax.experimental import pallas as pl
from jax.experimental.pallas import tpu as pltpu
from jax.experimental.pallas import tpu_sc as plsc

N = 10000
E = 320000
D = 128
NC = 2
NS = 16
NW = NC * NS
E_PER_W = E // NW
CHUNK = 100
CPB = 50
NBLK = E_PER_W // (CPB * CHUNK)
ROW_BLK = 624
LANES = 16


def _mm_kernel(x_ref, w_ref, o_ref):
    o_ref[...] = jnp.dot(x_ref[...], w_ref[...],
                         preferred_element_type=jnp.float32)


def _matmul(x, w):
    return pl.pallas_call(
        _mm_kernel,
        out_shape=jax.ShapeDtypeStruct((N, D), jnp.float32),
    )(x, w)


def _comb_mm_kernel(p_ref, w_ref, o_ref):
    s = p_ref[0] + p_ref[1]
    s = jnp.where(s >= 0, s, 0.01 * s)
    o_ref[...] = jnp.dot(s, w_ref[...], preferred_element_type=jnp.float32)


def _comb_matmul(p, w):
    return pl.pallas_call(
        _comb_mm_kernel,
        out_shape=jax.ShapeDtypeStruct((N, D), jnp.float32),
    )(p, w)


def _comb_kernel(p_ref, o_ref):
    s = p_ref[0] + p_ref[1]
    o_ref[...] = jnp.where(s >= 0, s, 0.01 * s)


def _combine(p):
    return pl.pallas_call(
        _comb_kernel,
        out_shape=jax.ShapeDtypeStruct((N, D), jnp.float32),
    )(p)


_sc_mesh = plsc.VectorSubcoreMesh(
    core_axis_name="c", subcore_axis_name="s", num_cores=NC, num_subcores=NS)


def _scale_rows(rows, ew, j):
    for b in range(0, (CHUNK // LANES) * LANES, LANES):
        wv = ew[j, pl.ds(b, LANES)]
        for l in range(LANES):
            w = wv[l]
            for d in range(D // LANES):
                slc = pl.ds(d * LANES, LANES)
                rows[b + l, slc] = rows[b + l, slc] * w
    rem = CHUNK % LANES
    if rem:
        base = CHUNK - LANES
        wv = ew[j, pl.ds(base, LANES)]
        for l in range(LANES - rem, LANES):
            w = wv[l]
            for d in range(D // LANES):
                slc = pl.ds(d * LANES, LANES)
                rows[base + l, slc] = rows[base + l, slc] * w


@functools.partial(
    pl.kernel,
    out_type=jax.ShapeDtypeStruct((NC, N, D), jnp.float32),
    mesh=_sc_mesh,
    scratch_types=[
        pltpu.VMEM_SHARED((N, D), jnp.float32),
        pltpu.VMEM((CHUNK, D), jnp.float32),
        pltpu.VMEM((CPB, CHUNK), jnp.int32),
        pltpu.VMEM((CPB, CHUNK), jnp.int32),
        pltpu.VMEM((CPB, CHUNK), jnp.float32),
        pltpu.SemaphoreType.DMA,
    ],
)
def _sc_scatter(h_hbm, src_hbm, dst_hbm, ew_hbm, out_hbm,
                acc, rows, sidx, didx, ew, sem):
    c = lax.axis_index("c")
    s = lax.axis_index("s")
    w = c * NS + s

    @pl.loop(0, CHUNK)
    def _zero_rows(i):
        for d in range(D // LANES):
            rows[i, pl.ds(d * LANES, LANES)] = jnp.zeros((LANES,), jnp.float32)

    base_row = s * ROW_BLK
    for z in range(6):
        pltpu.sync_copy(rows, acc.at[pl.ds(base_row + z * CHUNK, CHUNK)])
    pltpu.sync_copy(rows, acc.at[pl.ds(base_row + 540, CHUNK)])
    plsc.subcore_barrier()

    for blk in range(NBLK):
        pltpu.sync_copy(src_hbm.at[w].at[blk], sidx)
        pltpu.sync_copy(dst_hbm.at[w].at[blk], didx)
        pltpu.sync_copy(ew_hbm.at[w].at[blk], ew)

        @pl.loop(0, CPB)
        def _chunk(j):
            pltpu.async_copy(h_hbm.at[sidx.at[j]], rows, sem).wait()
            _scale_rows(rows, ew, j)
            pltpu.async_copy(rows, acc.at[didx.at[j]], sem, add=True).wait()

    plsc.subcore_barrier()
    pltpu.sync_copy(acc.at[pl.ds(base_row, 640)],
                    out_hbm.at[c].at[pl.ds(base_row, 640)])


def kernel(x, edge_index, edge_w, W0, W1):
    src = edge_index[0].astype(jnp.int32).reshape(NW, NBLK, CPB, CHUNK)
    dst = edge_index[1].astype(jnp.int32).reshape(NW, NBLK, CPB, CHUNK)
    ew = edge_w.reshape(NW, NBLK, CPB, CHUNK)
    h0 = _matmul(x, W0)
    p0 = _sc_scatter(h0, src, dst, ew)
    h1 = _comb_matmul(p0, W1)
    p1 = _sc_scatter(h1, src, dst, ew)
    return _combine(p1)


# speedup vs baseline: 7.6179x; 1.2124x over previous
import functools

import jax
import jax.numpy as jnp
from jax import lax
from jax.experimental import pallas as pl
from jax.experimental.pallas import tpu as pltpu
from jax.experimental.pallas import tpu_sc as plsc

N = 10000
E = 320000
D = 128
NC = 2
NS = 16
NW = NC * NS
E_PER_W = E // NW
CHUNK = 100
CPB = 50
NBLK = E_PER_W // (CPB * CHUNK)
ROW_BLK = 624
LANES = 16


def _mm_kernel(x_ref, w_ref, o_ref):
    o_ref[...] = jnp.dot(x_ref[...], w_ref[...],
                         preferred_element_type=jnp.float32)


def _matmul(x, w):
    return pl.pallas_call(
        _mm_kernel,
        out_shape=jax.ShapeDtypeStruct((N, D), jnp.float32),
    )(x, w)


def _comb_mm_kernel(p_ref, w_ref, o_ref):
    s = p_ref[0] + p_ref[1]
    s = jnp.where(s >= 0, s, 0.01 * s)
    o_ref[...] = jnp.dot(s, w_ref[...], preferred_element_type=jnp.float32)


def _comb_matmul(p, w):
    return pl.pallas_call(
        _comb_mm_kernel,
        out_shape=jax.ShapeDtypeStruct((N, D), jnp.float32),
    )(p, w)


def _comb_kernel(p_ref, o_ref):
    s = p_ref[0] + p_ref[1]
    o_ref[...] = jnp.where(s >= 0, s, 0.01 * s)


def _combine(p):
    return pl.pallas_call(
        _comb_kernel,
        out_shape=jax.ShapeDtypeStruct((N, D), jnp.float32),
    )(p)


_sc_mesh = plsc.VectorSubcoreMesh(
    core_axis_name="c", subcore_axis_name="s", num_cores=NC, num_subcores=NS)


def _scale_rows(rows, ew, j):
    for b in range(0, (CHUNK // LANES) * LANES, LANES):
        wv = ew[j, pl.ds(b, LANES)]
        for l in range(LANES):
            w = wv[l]
            for d in range(D // LANES):
                slc = pl.ds(d * LANES, LANES)
                rows[b + l, slc] = rows[b + l, slc] * w
    rem = CHUNK % LANES
    if rem:
        base = CHUNK - LANES
        wv = ew[j, pl.ds(base, LANES)]
        for l in range(LANES - rem, LANES):
            w = wv[l]
            for d in range(D // LANES):
                slc = pl.ds(d * LANES, LANES)
                rows[base + l, slc] = rows[base + l, slc] * w


@functools.partial(
    pl.kernel,
    out_type=jax.ShapeDtypeStruct((NC, N, D), jnp.float32),
    mesh=_sc_mesh,
    scratch_types=[
        pltpu.VMEM_SHARED((N, D), jnp.float32),
        pltpu.VMEM((CHUNK, D), jnp.float32),
        pltpu.VMEM((CHUNK, D), jnp.float32),
        pltpu.VMEM((CPB, CHUNK), jnp.int32),
        pltpu.VMEM((CPB, CHUNK), jnp.int32),
        pltpu.VMEM((CPB, CHUNK), jnp.float32),
        pltpu.SemaphoreType.DMA,
        pltpu.SemaphoreType.DMA,
        pltpu.SemaphoreType.DMA,
        pltpu.SemaphoreType.DMA,
    ],
)
def _sc_scatter(h_hbm, src_hbm, dst_hbm, ew_hbm, out_hbm,
                acc, rows0, rows1, sidx, didx, ew,
                gsem0, gsem1, ssem0, ssem1):
    c = lax.axis_index("c")
    s = lax.axis_index("s")
    w = c * NS + s

    @pl.loop(0, CHUNK)
    def _zero_rows(i):
        for d in range(D // LANES):
            rows0[i, pl.ds(d * LANES, LANES)] = jnp.zeros((LANES,), jnp.float32)

    base_row = s * ROW_BLK
    for z in range(6):
        pltpu.sync_copy(rows0, acc.at[pl.ds(base_row + z * CHUNK, CHUNK)])
    pltpu.sync_copy(rows0, acc.at[pl.ds(base_row + 540, CHUNK)])
    plsc.subcore_barrier()

    for blk in range(NBLK):
        pltpu.sync_copy(src_hbm.at[w].at[blk], sidx)
        pltpu.sync_copy(dst_hbm.at[w].at[blk], didx)
        pltpu.sync_copy(ew_hbm.at[w].at[blk], ew)

        pltpu.async_copy(h_hbm.at[sidx.at[0]], rows0, gsem0)

        @pl.loop(0, CPB, step=2)
        def _pair(j):
            pltpu.make_async_copy(h_hbm.at[sidx.at[j]], rows0, gsem0).wait()

            @pl.when(j > 0)
            def _wait_s1():
                pltpu.make_async_copy(
                    rows1, acc.at[didx.at[j - 1]], ssem1).wait()

            pltpu.async_copy(h_hbm.at[sidx.at[j + 1]], rows1, gsem1)
            _scale_rows(rows0, ew, j)
            pltpu.async_copy(rows0, acc.at[didx.at[j]], ssem0, add=True)

            pltpu.make_async_copy(h_hbm.at[sidx.at[j + 1]], rows1, gsem1).wait()
            pltpu.make_async_copy(rows0, acc.at[didx.at[j]], ssem0).wait()

            @pl.when(j + 2 < CPB)
            def _next_gather():
                pltpu.async_copy(h_hbm.at[sidx.at[j + 2]], rows0, gsem0)

            _scale_rows(rows1, ew, j + 1)
            pltpu.async_copy(rows1, acc.at[didx.at[j + 1]], ssem1, add=True)

        pltpu.make_async_copy(rows1, acc.at[didx.at[CPB - 1]], ssem1).wait()

    plsc.subcore_barrier()
    pltpu.sync_copy(acc.at[pl.ds(base_row, 640)],
                    out_hbm.at[c].at[pl.ds(base_row, 640)])


def kernel(x, edge_index, edge_w, W0, W1):
    src = edge_index[0].astype(jnp.int32).reshape(NW, NBLK, CPB, CHUNK)
    dst = edge_index[1].astype(jnp.int32).reshape(NW, NBLK, CPB, CHUNK)
    ew = edge_w.reshape(NW, NBLK, CPB, CHUNK)
    h0 = _matmul(x, W0)
    p0 = _sc_scatter(h0, src, dst, ew)
    h1 = _comb_matmul(p0, W1)
    p1 = _sc_scatter(h1, src, dst, ew)
    return _combine(p1)


# speedup vs baseline: 7.6474x; 1.0039x over previous
import functools

import jax
import jax.numpy as jnp
from jax import lax
from jax.experimental import pallas as pl
from jax.experimental.pallas import tpu as pltpu
from jax.experimental.pallas import tpu_sc as plsc

N = 10000
E = 320000
D = 128
NC = 2
NS = 16
NW = NC * NS
E_PER_W = E // NW
CHUNK = 100
CPB = 50
NBLK = E_PER_W // (CPB * CHUNK)
ROW_BLK = 624
LANES = 16


def _mm_kernel(x_ref, w_ref, o_ref):
    o_ref[...] = jnp.dot(x_ref[...], w_ref[...],
                         preferred_element_type=jnp.float32)


def _matmul(x, w):
    return pl.pallas_call(
        _mm_kernel,
        out_shape=jax.ShapeDtypeStruct((N, D), jnp.float32),
    )(x, w)


def _comb_mm_kernel(p_ref, w_ref, o_ref):
    s = p_ref[0] + p_ref[1]
    s = jnp.where(s >= 0, s, 0.01 * s)
    o_ref[...] = jnp.dot(s, w_ref[...], preferred_element_type=jnp.float32)


def _comb_matmul(p, w):
    return pl.pallas_call(
        _comb_mm_kernel,
        out_shape=jax.ShapeDtypeStruct((N, D), jnp.float32),
    )(p, w)


def _comb_kernel(p_ref, o_ref):
    s = p_ref[0] + p_ref[1]
    o_ref[...] = jnp.where(s >= 0, s, 0.01 * s)


def _combine(p):
    return pl.pallas_call(
        _comb_kernel,
        out_shape=jax.ShapeDtypeStruct((N, D), jnp.float32),
    )(p)


_sc_mesh = plsc.VectorSubcoreMesh(
    core_axis_name="c", subcore_axis_name="s", num_cores=NC, num_subcores=NS)


def _bcast_lane(wv, l):
    return wv.at[jnp.full((LANES,), l, jnp.int32)].get(
        mode="promise_in_bounds")


def _scale_rows(rows, ew, j):
    for b in range(0, (CHUNK // LANES) * LANES, LANES):
        wv = ew[j, pl.ds(b, LANES)]
        for l in range(LANES):
            w = _bcast_lane(wv, l)
            for d in range(D // LANES):
                slc = pl.ds(d * LANES, LANES)
                rows[b + l, slc] = rows[b + l, slc] * w
    rem = CHUNK % LANES
    if rem:
        base = CHUNK - LANES
        wv = ew[j, pl.ds(base, LANES)]
        for l in range(LANES - rem, LANES):
            w = _bcast_lane(wv, l)
            for d in range(D // LANES):
                slc = pl.ds(d * LANES, LANES)
                rows[base + l, slc] = rows[base + l, slc] * w


@functools.partial(
    pl.kernel,
    out_type=jax.ShapeDtypeStruct((NC, N, D), jnp.float32),
    mesh=_sc_mesh,
    scratch_types=[
        pltpu.VMEM_SHARED((N, D), jnp.float32),
        pltpu.VMEM((CHUNK, D), jnp.float32),
        pltpu.VMEM((CHUNK, D), jnp.float32),
        pltpu.VMEM((CPB, CHUNK), jnp.int32),
        pltpu.VMEM((CPB, CHUNK), jnp.int32),
        pltpu.VMEM((CPB, CHUNK), jnp.float32),
        pltpu.SemaphoreType.DMA,
        pltpu.SemaphoreType.DMA,
        pltpu.SemaphoreType.DMA,
        pltpu.SemaphoreType.DMA,
    ],
)
def _sc_scatter(h_hbm, src_hbm, dst_hbm, ew_hbm, out_hbm,
                acc, rows0, rows1, sidx, didx, ew,
                gsem0, gsem1, ssem0, ssem1):
    c = lax.axis_index("c")
    s = lax.axis_index("s")
    w = c * NS + s

    @pl.loop(0, CHUNK)
    def _zero_rows(i):
        for d in range(D // LANES):
            rows0[i, pl.ds(d * LANES, LANES)] = jnp.zeros((LANES,), jnp.float32)

    base_row = s * ROW_BLK
    for z in range(6):
        pltpu.sync_copy(rows0, acc.at[pl.ds(base_row + z * CHUNK, CHUNK)])
    pltpu.sync_copy(rows0, acc.at[pl.ds(base_row + 540, CHUNK)])
    plsc.subcore_barrier()

    for blk in range(NBLK):
        pltpu.sync_copy(src_hbm.at[w].at[blk], sidx)
        pltpu.sync_copy(dst_hbm.at[w].at[blk], didx)
        pltpu.sync_copy(ew_hbm.at[w].at[blk], ew)

        pltpu.async_copy(h_hbm.at[sidx.at[0]], rows0, gsem0)

        @pl.loop(0, CPB, step=2)
        def _pair(j):
            pltpu.make_async_copy(h_hbm.at[sidx.at[j]], rows0, gsem0).wait()

            @pl.when(j > 0)
            def _wait_s1():
                pltpu.make_async_copy(
                    rows1, acc.at[didx.at[j - 1]], ssem1).wait()

            pltpu.async_copy(h_hbm.at[sidx.at[j + 1]], rows1, gsem1)
            _scale_rows(rows0, ew, j)
            pltpu.async_copy(rows0, acc.at[didx.at[j]], ssem0, add=True)

            pltpu.make_async_copy(h_hbm.at[sidx.at[j + 1]], rows1, gsem1).wait()
            pltpu.make_async_copy(rows0, acc.at[didx.at[j]], ssem0).wait()

            @pl.when(j + 2 < CPB)
            def _next_gather():
                pltpu.async_copy(h_hbm.at[sidx.at[j + 2]], rows0, gsem0)

            _scale_rows(rows1, ew, j + 1)
            pltpu.async_copy(rows1, acc.at[didx.at[j + 1]], ssem1, add=True)

        pltpu.make_async_copy(rows1, acc.at[didx.at[CPB - 1]], ssem1).wait()

    plsc.subcore_barrier()
    pltpu.sync_copy(acc.at[pl.ds(base_row, 640)],
                    out_hbm.at[c].at[pl.ds(base_row, 640)])


def kernel(x, edge_index, edge_w, W0, W1):
    src = edge_index[0].astype(jnp.int32).reshape(NW, NBLK, CPB, CHUNK)
    dst = edge_index[1].astype(jnp.int32).reshape(NW, NBLK, CPB, CHUNK)
    ew = edge_w.reshape(NW, NBLK, CPB, CHUNK)
    h0 = _matmul(x, W0)
    p0 = _sc_scatter(h0, src, dst, ew)
    h1 = _comb_matmul(p0, W1)
    p1 = _sc_scatter(h1, src, dst, ew)
    return _combine(p1)


# speedup vs baseline: 10.3758x; 1.3568x over previous
import functools

import jax
import jax.numpy as jnp
from jax import lax
from jax.experimental import pallas as pl
from jax.experimental.pallas import tpu as pltpu
from jax.experimental.pallas import tpu_sc as plsc

N = 10000
E = 320000
D = 128
NC = 2
NS = 16
NW = NC * NS
E_PER_W = E // NW
CHUNK = 80
CPB = 25
NBLK = E_PER_W // (CPB * CHUNK)
ROW_BLK = 624
LANES = 16
NBUF = 3


def _mm_kernel(x_ref, w_ref, o_ref):
    o_ref[...] = jnp.dot(x_ref[...], w_ref[...],
                         preferred_element_type=jnp.float32)


def _matmul(x, w):
    return pl.pallas_call(
        _mm_kernel,
        out_shape=jax.ShapeDtypeStruct((N, D), jnp.float32),
    )(x, w)


def _comb_mm_kernel(p_ref, w_ref, o_ref):
    s = p_ref[0] + p_ref[1]
    s = jnp.where(s >= 0, s, 0.01 * s)
    o_ref[...] = jnp.dot(s, w_ref[...], preferred_element_type=jnp.float32)


def _comb_matmul(p, w):
    return pl.pallas_call(
        _comb_mm_kernel,
        out_shape=jax.ShapeDtypeStruct((N, D), jnp.float32),
    )(p, w)


def _comb_kernel(p_ref, o_ref):
    s = p_ref[0] + p_ref[1]
    o_ref[...] = jnp.where(s >= 0, s, 0.01 * s)


def _combine(p):
    return pl.pallas_call(
        _comb_kernel,
        out_shape=jax.ShapeDtypeStruct((N, D), jnp.float32),
    )(p)


_sc_mesh = plsc.VectorSubcoreMesh(
    core_axis_name="c", subcore_axis_name="s", num_cores=NC, num_subcores=NS)


def _bcast_lane(wv, l):
    return wv.at[jnp.full((LANES,), l, jnp.int32)].get(
        mode="promise_in_bounds")


def _scale_rows(rows, ew, j):
    @pl.loop(0, CHUNK, step=LANES)
    def _group(b):
        wv = ew[j, pl.ds(b, LANES)]
        for l in range(LANES):
            w = _bcast_lane(wv, l)
            for d in range(D // LANES):
                slc = pl.ds(d * LANES, LANES)
                rows[b + l, slc] = rows[b + l, slc] * w


@functools.partial(
    pl.kernel,
    out_type=jax.ShapeDtypeStruct((NC, N, D), jnp.float32),
    mesh=_sc_mesh,
    scratch_types=[
        pltpu.VMEM_SHARED((N, D), jnp.float32),
        pltpu.VMEM((CHUNK, D), jnp.float32),
        pltpu.VMEM((CHUNK, D), jnp.float32),
        pltpu.VMEM((CHUNK, D), jnp.float32),
        pltpu.VMEM((CPB, CHUNK), jnp.int32),
        pltpu.VMEM((CPB, CHUNK), jnp.int32),
        pltpu.VMEM((CPB, CHUNK), jnp.float32),
        pltpu.SemaphoreType.DMA,
        pltpu.SemaphoreType.DMA,
        pltpu.SemaphoreType.DMA,
        pltpu.SemaphoreType.DMA,
        pltpu.SemaphoreType.DMA,
        pltpu.SemaphoreType.DMA,
    ],
)
def _sc_scatter(h_hbm, src_hbm, dst_hbm, ew_hbm, out_hbm,
                acc, rows0, rows1, rows2, sidx, didx, ew,
                gsem0, gsem1, gsem2, ssem0, ssem1, ssem2):
    c = lax.axis_index("c")
    s = lax.axis_index("s")
    w = c * NS + s
    rows = (rows0, rows1, rows2)
    gsem = (gsem0, gsem1, gsem2)
    ssem = (ssem0, ssem1, ssem2)

    @pl.loop(0, CHUNK)
    def _zero_rows(i):
        for d in range(D // LANES):
            rows0[i, pl.ds(d * LANES, LANES)] = jnp.zeros((LANES,), jnp.float32)

    base_row = s * ROW_BLK
    for z in range(8):
        pltpu.sync_copy(rows0, acc.at[pl.ds(base_row + z * CHUNK, CHUNK)])
    plsc.subcore_barrier()

    def wait_gather(k, b):
        pltpu.make_async_copy(h_hbm.at[sidx.at[k]], rows[b], gsem[b]).wait()

    def issue_gather(k, b):
        pltpu.async_copy(h_hbm.at[sidx.at[k]], rows[b], gsem[b])

    def wait_scatter(k, b):
        pltpu.make_async_copy(rows[b], acc.at[didx.at[k]], ssem[b]).wait()

    def issue_scatter(k, b):
        pltpu.async_copy(rows[b], acc.at[didx.at[k]], ssem[b], add=True)

    for blk in range(NBLK):
        pltpu.sync_copy(src_hbm.at[w].at[blk], sidx)
        pltpu.sync_copy(dst_hbm.at[w].at[blk], didx)
        pltpu.sync_copy(ew_hbm.at[w].at[blk], ew)

        issue_gather(0, 0)
        issue_gather(1, 1)

        @pl.loop(0, CPB - 1, step=NBUF)
        def _tri(j):
            for o in range(NBUF):
                k = j + o
                b = o % NBUF
                bn = (o + 2) % NBUF
                wait_gather(k, b)
                _scale_rows(rows[b], ew, k)
                issue_scatter(k, b)
                if o >= 1:
                    wait_scatter(k - 1, bn)
                else:
                    @pl.when(j > 0)
                    def _w():
                        wait_scatter(k - 1, bn)
                if o < 2:
                    issue_gather(k + 2, bn)
                else:
                    @pl.when(j < CPB - 1 - NBUF)
                    def _g():
                        issue_gather(k + 2, bn)

        wait_gather(CPB - 1, 0)
        _scale_rows(rows[0], ew, CPB - 1)
        issue_scatter(CPB - 1, 0)
        wait_scatter(CPB - 2, 2)
        wait_scatter(CPB - 1, 0)

    plsc.subcore_barrier()
    pltpu.sync_copy(acc.at[pl.ds(base_row, 640)],
                    out_hbm.at[c].at[pl.ds(base_row, 640)])


def kernel(x, edge_index, edge_w, W0, W1):
    src = edge_index[0].astype(jnp.int32).reshape(NW, NBLK, CPB, CHUNK)
    dst = edge_index[1].astype(jnp.int32).reshape(NW, NBLK, CPB, CHUNK)
    ew = edge_w.reshape(NW, NBLK, CPB, CHUNK)
    h0 = _matmul(x, W0)
    p0 = _sc_scatter(h0, src, dst, ew)
    h1 = _comb_matmul(p0, W1)
    p1 = _sc_scatter(h1, src, dst, ew)
    return _combine(p1)
